# baseline (speedup 1.0000x reference)
import functools

import jax
import jax.numpy as jnp
from jax import lax
from jax.experimental import pallas as pl
from jax.experimental.pallas import tpu as pltpu
from jax.experimental.pallas import tpu_sc as plsc

N_E = 1024
D = 64
R = 512
G = 9
N_ROWS = R * G
NW = 32
B_PER_W = N_ROWS // NW
CHUNK = 72
N_CHUNKS = B_PER_W // CHUNK
LOSS_SCALE = 0.5 / float(N_ROWS * D)


def _vq_body(x_ref, w_ref, idx_ref, loss_ref):
    x = x_ref[...]
    w = w_ref[...]
    num = lax.dot_general(x, w, (((1,), (1,)), ((), ())))
    xn2 = jnp.sum(x * x, axis=1, keepdims=True)
    xn = jnp.sqrt(xn2)
    wn2 = jnp.sum(w * w, axis=1)
    wn = jnp.sqrt(wn2)
    denom = jnp.maximum(xn * wn[None, :], 1e-8)
    dist = 1.0 - num / denom
    dmin = jnp.min(dist, axis=1, keepdims=True)
    mask = dist == dmin
    col = lax.broadcasted_iota(jnp.int32, (R, N_E), 1)
    idx = jnp.min(jnp.where(mask, col, N_E), axis=1)
    idx_ref[...] = idx

    tsel = jnp.max(jnp.where(mask, 2.0 * num - wn2[None, :], -jnp.inf), axis=1)
    part = (LOSS_SCALE * (jnp.sum(xn2) - jnp.sum(tsel))).reshape(1, 1)

    @pl.when(pl.program_id(0) == 0)
    def _init():
        loss_ref[...] = jnp.zeros((1, 1), jnp.float32)

    loss_ref[...] += part


_vq_call = pl.pallas_call(
    _vq_body,
    grid=(G,),
    in_specs=[
        pl.BlockSpec((R, D), lambda i: (i, 0)),
        pl.BlockSpec((N_E, D), lambda i: (0, 0)),
    ],
    out_specs=[
        pl.BlockSpec((R,), lambda i: (i,)),
        pl.BlockSpec((1, 1), lambda i: (0, 0)),
    ],
    out_shape=[
        jax.ShapeDtypeStruct((N_ROWS,), jnp.int32),
        jax.ShapeDtypeStruct((1, 1), jnp.float32),
    ],
)


@functools.lru_cache(maxsize=1)
def _make_sc_gather():
    mesh = plsc.VectorSubcoreMesh(core_axis_name="c", subcore_axis_name="s")

    @functools.partial(
        pl.kernel,
        mesh=mesh,
        out_type=jax.ShapeDtypeStruct((N_ROWS, D), jnp.float32),
        compiler_params=pltpu.CompilerParams(use_tc_tiling_on_sc=False),
        scratch_types=[
            pltpu.VMEM((B_PER_W,), jnp.int32),
            pltpu.VMEM((B_PER_W, D), jnp.float32),
            pltpu.SemaphoreType.DMA,
        ],
    )
    def gather(table_hbm, idx_hbm, out_hbm, idx_v, rows_v, sem):
        wid = lax.axis_index("s") * 2 + lax.axis_index("c")
        base = wid * B_PER_W
        pltpu.sync_copy(idx_hbm.at[pl.ds(base, B_PER_W)], idx_v)
        copies = [
            pltpu.async_copy(
                table_hbm.at[idx_v.at[pl.ds(j * CHUNK, CHUNK)]],
                rows_v.at[pl.ds(j * CHUNK, CHUNK)],
                sem,
            )
            for j in range(N_CHUNKS)
        ]
        for c in copies:
            c.wait()
        pltpu.sync_copy(rows_v, out_hbm.at[pl.ds(base, B_PER_W)])

    return gather


def kernel(inputs, weight):
    flat = inputs.reshape(N_ROWS, D)
    idx_flat, loss_sum = _vq_call(flat, weight)
    q = _make_sc_gather()(weight, idx_flat)
    quantized = q.reshape(inputs.shape)
    loss = loss_sum[0, 0]
    return quantized, loss, idx_flat[:, None]


# speedup vs baseline: 1.0668x; 1.0668x over previous
import functools

import jax
import jax.numpy as jnp
from jax import lax
from jax.experimental import pallas as pl
from jax.experimental.pallas import tpu as pltpu
from jax.experimental.pallas import tpu_sc as plsc

N_E = 1024
D = 64
RB = 576
NB = 8
N_ROWS = RB * NB
NW = 32
B_PER_W = N_ROWS // NW
CHUNK = 72
N_CHUNKS = B_PER_W // CHUNK
LANES = 16
LOSS_SCALE = 0.5 / float(N_ROWS * D)


def _vq_body(x_ref, w_ref, idx_ref):
    w = w_ref[...]
    wn = jnp.sqrt(jnp.sum(w * w, axis=1))
    for b in range(NB):
        x = x_ref[pl.ds(b * RB, RB), :]
        num = lax.dot_general(x, w, (((1,), (1,)), ((), ())))
        xn = jnp.sqrt(jnp.sum(x * x, axis=1, keepdims=True))
        denom = jnp.maximum(xn * wn[None, :], 1e-8)
        dist = 1.0 - num / denom
        dmin = jnp.min(dist, axis=1, keepdims=True)
        col = lax.broadcasted_iota(jnp.int32, (RB, N_E), 1)
        idx = jnp.min(jnp.where(dist == dmin, col, N_E), axis=1)
        idx_ref[pl.ds(b * RB, RB)] = idx


_vq_call = pl.pallas_call(
    _vq_body,
    out_shape=jax.ShapeDtypeStruct((N_ROWS,), jnp.int32),
)


@functools.lru_cache(maxsize=1)
def _make_sc_gather():
    mesh = plsc.VectorSubcoreMesh(core_axis_name="c", subcore_axis_name="s")

    @functools.partial(
        pl.kernel,
        mesh=mesh,
        out_type=[
            jax.ShapeDtypeStruct((N_ROWS, D), jnp.float32),
            jax.ShapeDtypeStruct((NW, LANES), jnp.float32),
        ],
        compiler_params=pltpu.CompilerParams(use_tc_tiling_on_sc=False),
        scratch_types=[
            pltpu.VMEM((B_PER_W,), jnp.int32),
            pltpu.VMEM((B_PER_W, D), jnp.float32),
            pltpu.VMEM((B_PER_W, D), jnp.float32),
            pltpu.VMEM((LANES,), jnp.float32),
            pltpu.SemaphoreType.DMA,
            pltpu.SemaphoreType.DMA,
        ],
    )
    def gather(table_hbm, idx_hbm, x_hbm, out_hbm, part_hbm,
               idx_v, rows_v, x_v, part_v, sem_g, sem_x):
        wid = lax.axis_index("s") * 2 + lax.axis_index("c")
        base = wid * B_PER_W
        cp_x = pltpu.async_copy(x_hbm.at[pl.ds(base, B_PER_W)], x_v, sem_x)
        pltpu.sync_copy(idx_hbm.at[pl.ds(base, B_PER_W)], idx_v)
        gathers = [
            pltpu.async_copy(
                table_hbm.at[idx_v.at[pl.ds(j * CHUNK, CHUNK)]],
                rows_v.at[pl.ds(j * CHUNK, CHUNK)],
                sem_g,
            )
            for j in range(N_CHUNKS)
        ]
        for g in gathers:
            g.wait()
        pltpu.sync_copy(rows_v, out_hbm.at[pl.ds(base, B_PER_W)])
        cp_x.wait()

        def body(r, acc):
            for c in range(D // LANES):
                d = (x_v[r, pl.ds(c * LANES, LANES)]
                     - rows_v[r, pl.ds(c * LANES, LANES)])
                acc = acc + d * d
            return acc

        acc = lax.fori_loop(0, B_PER_W, body, jnp.zeros((LANES,), jnp.float32))
        part_v[...] = acc
        pltpu.sync_copy(part_v, part_hbm.at[wid])

    return gather


def kernel(inputs, weight):
    flat = inputs.reshape(N_ROWS, D)
    idx_flat = _vq_call(flat, weight)
    q, parts = _make_sc_gather()(weight, idx_flat, flat)
    quantized = q.reshape(inputs.shape)
    loss = jnp.sum(parts) * LOSS_SCALE
    return quantized, loss, idx_flat[:, None]


# speedup vs baseline: 1.0878x; 1.0197x over previous
import functools

import jax
import jax.numpy as jnp
from jax import lax
from jax.experimental import pallas as pl
from jax.experimental.pallas import tpu as pltpu
from jax.experimental.pallas import tpu_sc as plsc

N_E = 1024
D = 64
RB = 576
NB = 8
N_ROWS = RB * NB
NW = 32
B_PER_W = N_ROWS // NW
CHUNK = 72
N_CHUNKS = B_PER_W // CHUNK
LANES = 16
LOSS_SCALE = 0.5 / float(N_ROWS * D)


def _vq_body(x_ref, w_ref, idx_ref, loss_ref):
    w = w_ref[...]
    wn2 = jnp.sum(w * w, axis=1)
    wn = jnp.sqrt(wn2)
    total = jnp.zeros((1, 1), jnp.float32)
    for b in range(NB):
        x = x_ref[pl.ds(b * RB, RB), :]
        num = lax.dot_general(x, w, (((1,), (1,)), ((), ())))
        xn2 = jnp.sum(x * x, axis=1, keepdims=True)
        xn = jnp.sqrt(xn2)
        denom = jnp.maximum(xn * wn[None, :], 1e-8)
        dist = 1.0 - num / denom
        dmin = jnp.min(dist, axis=1, keepdims=True)
        mask = dist == dmin
        col = lax.broadcasted_iota(jnp.int32, (RB, N_E), 1)
        idx = jnp.min(jnp.where(mask, col, N_E), axis=1)
        idx_ref[pl.ds(b * RB, RB)] = idx
        tsel = jnp.max(jnp.where(mask, 2.0 * num - wn2[None, :], -jnp.inf),
                       axis=1)
        total += (jnp.sum(xn2) - jnp.sum(tsel)).reshape(1, 1)
    loss_ref[...] = total * LOSS_SCALE


_vq_call = pl.pallas_call(
    _vq_body,
    out_shape=[
        jax.ShapeDtypeStruct((N_ROWS,), jnp.int32),
        jax.ShapeDtypeStruct((1, 1), jnp.float32),
    ],
)


@functools.lru_cache(maxsize=1)
def _make_sc_gather():
    mesh = plsc.VectorSubcoreMesh(core_axis_name="c", subcore_axis_name="s")

    @functools.partial(
        pl.kernel,
        mesh=mesh,
        out_type=jax.ShapeDtypeStruct((N_ROWS, D), jnp.float32),
        compiler_params=pltpu.CompilerParams(use_tc_tiling_on_sc=False),
        scratch_types=[
            pltpu.VMEM((B_PER_W,), jnp.int32),
            pltpu.VMEM((B_PER_W, D), jnp.float32),
            pltpu.SemaphoreType.DMA,
        ],
    )
    def gather(table_hbm, idx_hbm, out_hbm, idx_v, rows_v, sem_g):
        wid = lax.axis_index("s") * 2 + lax.axis_index("c")
        base = wid * B_PER_W
        pltpu.sync_copy(idx_hbm.at[pl.ds(base, B_PER_W)], idx_v)
        gathers = [
            pltpu.async_copy(
                table_hbm.at[idx_v.at[pl.ds(j * CHUNK, CHUNK)]],
                rows_v.at[pl.ds(j * CHUNK, CHUNK)],
                sem_g,
            )
            for j in range(N_CHUNKS)
        ]
        for g in gathers:
            g.wait()
        pltpu.sync_copy(rows_v, out_hbm.at[pl.ds(base, B_PER_W)])

    return gather


def kernel(inputs, weight):
    flat = inputs.reshape(N_ROWS, D)
    idx_flat, loss_sum = _vq_call(flat, weight)
    q = _make_sc_gather()(weight, idx_flat)
    quantized = q.reshape(inputs.shape)
    loss = loss_sum[0, 0]
    return quantized, loss, idx_flat[:, None]
